# baseline (speedup 1.0000x reference)
import functools

import jax
import jax.numpy as jnp
from jax import lax
from jax.experimental import pallas as pl
from jax.experimental.pallas import tpu as pltpu
from jax.experimental.pallas import tpu_sc as plsc

_D = 64
_B = 4096
_H = 50
_NW = 32
_BPW = _B // _NW
_NBUF = 16
_NL = 16
_DBLK = _D // _NL
_INV_H = 1.0 / _H


def _sc_body(text_hbm, table_hbm, out_hbm, idx_v, rows_v, out_v, sems):
    wid = lax.axis_index("s") * 2 + lax.axis_index("c")
    base = wid * _BPW

    pltpu.sync_copy(text_hbm.at[pl.ds(base, _BPW)], idx_v)

    def _fire(r, b):
        pltpu.make_async_copy(
            table_hbm.at[idx_v.at[r]], rows_v.at[b], sems.at[b]
        ).start()

    def _wait(b):
        pltpu.make_async_copy(
            table_hbm.at[idx_v.at[0]], rows_v.at[b], sems.at[b]
        ).wait()

    for b in range(_NBUF):
        _fire(b, b)

    def _outer(g, carry):
        r0 = g * _NBUF
        for b in range(_NBUF):
            r = r0 + b
            _wait(b)
            rbuf = rows_v.at[b]

            def _jbody(j, accs, rbuf=rbuf):
                a = tuple(
                    accs[k] + rbuf[2 * j, pl.ds(_NL * k, _NL)]
                    for k in range(_DBLK)
                )
                return tuple(
                    a[k] + rbuf[2 * j + 1, pl.ds(_NL * k, _NL)]
                    for k in range(_DBLK)
                )

            z = jnp.zeros((_NL,), jnp.float32)
            accs = lax.fori_loop(0, _H // 2, _jbody, (z,) * _DBLK)

            nxt = r + _NBUF

            @pl.when(nxt < _BPW)
            def _():
                _fire(nxt, b)

            for k in range(_DBLK):
                out_v[r, pl.ds(_NL * k, _NL)] = accs[k] * _INV_H
        return carry

    lax.fori_loop(0, _BPW // _NBUF, _outer, 0)

    pltpu.sync_copy(out_v, out_hbm.at[pl.ds(base, _BPW)])


@functools.partial(
    pl.kernel,
    out_type=jax.ShapeDtypeStruct((_B, _D), jnp.float32),
    mesh=plsc.VectorSubcoreMesh(core_axis_name="c", subcore_axis_name="s"),
    compiler_params=pltpu.CompilerParams(use_tc_tiling_on_sc=False),
    scratch_types=[
        pltpu.VMEM((_BPW, _H), jnp.int32),
        pltpu.VMEM((_NBUF, _H, _D), jnp.float32),
        pltpu.VMEM((_BPW, _D), jnp.float32),
        pltpu.SemaphoreType.DMA((_NBUF,)),
    ],
)
def _embed_mean(text_hbm, table_hbm, out_hbm, idx_v, rows_v, out_v, sems):
    _sc_body(text_hbm, table_hbm, out_hbm, idx_v, rows_v, out_v, sems)


def kernel(text, text_length, embeddings):
    del text_length
    return _embed_mean(text.astype(jnp.int32), embeddings)


# speedup vs baseline: 1.0064x; 1.0064x over previous
import functools

import jax
import jax.numpy as jnp
from jax import lax
from jax.experimental import pallas as pl
from jax.experimental.pallas import tpu as pltpu
from jax.experimental.pallas import tpu_sc as plsc

_D = 64
_B = 4096
_H = 50
_NW = 32
_BPW = _B // _NW
_NBUF = 8
_NL = 16
_DBLK = _D // _NL
_INV_H = 1.0 / _H


def _sc_body(text_hbm, table_hbm, out_hbm, idx_v, rows_v, out_v, sems):
    wid = lax.axis_index("s") * 2 + lax.axis_index("c")
    base = wid * _BPW

    pltpu.sync_copy(text_hbm.at[pl.ds(base, _BPW)], idx_v)

    def _fire(r, b):
        pltpu.make_async_copy(
            table_hbm.at[idx_v.at[r]], rows_v.at[b], sems.at[b]
        ).start()

    def _wait(b):
        pltpu.make_async_copy(
            table_hbm.at[idx_v.at[0]], rows_v.at[b], sems.at[b]
        ).wait()

    for b in range(_NBUF):
        _fire(b, b)

    def _outer(g, carry):
        r0 = g * _NBUF
        for b in range(_NBUF):
            r = r0 + b
            _wait(b)
            rbuf = rows_v.at[b]

            def _jbody(j, accs, rbuf=rbuf):
                a = tuple(
                    accs[k] + rbuf[2 * j, pl.ds(_NL * k, _NL)]
                    for k in range(_DBLK)
                )
                return tuple(
                    a[k] + rbuf[2 * j + 1, pl.ds(_NL * k, _NL)]
                    for k in range(_DBLK)
                )

            z = jnp.zeros((_NL,), jnp.float32)
            accs = lax.fori_loop(0, _H // 2, _jbody, (z,) * _DBLK)

            nxt = r + _NBUF

            @pl.when(nxt < _BPW)
            def _():
                _fire(nxt, b)

            for k in range(_DBLK):
                out_v[r, pl.ds(_NL * k, _NL)] = accs[k] * _INV_H
        return carry

    lax.fori_loop(0, _BPW // _NBUF, _outer, 0)

    pltpu.sync_copy(out_v, out_hbm.at[pl.ds(base, _BPW)])


@functools.partial(
    pl.kernel,
    out_type=jax.ShapeDtypeStruct((_B, _D), jnp.float32),
    mesh=plsc.VectorSubcoreMesh(core_axis_name="c", subcore_axis_name="s"),
    compiler_params=pltpu.CompilerParams(use_tc_tiling_on_sc=False),
    scratch_types=[
        pltpu.VMEM((_BPW, _H), jnp.int32),
        pltpu.VMEM((_NBUF, _H, _D), jnp.float32),
        pltpu.VMEM((_BPW, _D), jnp.float32),
        pltpu.SemaphoreType.DMA((_NBUF,)),
    ],
)
def _embed_mean(text_hbm, table_hbm, out_hbm, idx_v, rows_v, out_v, sems):
    _sc_body(text_hbm, table_hbm, out_hbm, idx_v, rows_v, out_v, sems)


def kernel(text, text_length, embeddings):
    del text_length
    return _embed_mean(text.astype(jnp.int32), embeddings)
